# Initial kernel scaffold; baseline (speedup 1.0000x reference)
import jax
import jax.numpy as jnp
from jax.experimental import pallas as pl


def kernel(x, pos, edge_index, batch, params):
    raise NotImplementedError("write your pallas kernel here")


# speedup vs baseline: 4.1180x; 4.1180x over previous
import functools

import jax
import jax.numpy as jnp
from jax import lax
from jax.experimental import pallas as pl
from jax.experimental.pallas import tpu as pltpu
from jax.experimental.pallas import tpu_sc as plsc

N = 10000
E = 320000
H = 64
HP = 128
B = 64
L = 7

NC = 2
NS = 16
NW = NC * NS
N_PAD = 10240
ROWS_PER_TILE = N_PAD // NS
EDGES_PER_WORKER = E // NW
CHUNK = 80
NCHUNK = EDGES_PER_WORKER // CHUNK


def _segsum_body(hp_hbm, src_hbm, dst_hbm, zero_hbm, out_hbm,
                 sidx, didx, rows, acc, sem):
    c = lax.axis_index("c")
    s = lax.axis_index("s")
    wid = c * NS + s
    r0 = s * ROWS_PER_TILE
    pltpu.sync_copy(zero_hbm.at[pl.ds(r0, ROWS_PER_TILE)],
                    acc.at[pl.ds(r0, ROWS_PER_TILE)])
    plsc.subcore_barrier()

    base = wid * EDGES_PER_WORKER

    def body(j, carry):
        off = base + j * CHUNK
        pltpu.sync_copy(src_hbm.at[pl.ds(off, CHUNK)], sidx)
        pltpu.sync_copy(dst_hbm.at[pl.ds(off, CHUNK)], didx)
        pltpu.async_copy(hp_hbm.at[sidx], rows, sem).wait()
        pltpu.sync_copy(rows, acc.at[didx], add=True)
        return carry

    lax.fori_loop(0, NCHUNK, body, 0)
    plsc.subcore_barrier()
    pltpu.sync_copy(acc.at[pl.ds(r0, ROWS_PER_TILE)],
                    out_hbm.at[c, pl.ds(r0, ROWS_PER_TILE)])


@functools.cache
def _build_segsum():
    return functools.partial(
        pl.kernel,
        out_type=jax.ShapeDtypeStruct((NC, N_PAD, HP), jnp.float32),
        mesh=plsc.VectorSubcoreMesh(core_axis_name="c", subcore_axis_name="s",
                                    num_cores=NC, num_subcores=NS),
        scratch_types=[
            pltpu.VMEM((CHUNK,), jnp.int32),
            pltpu.VMEM((CHUNK,), jnp.int32),
            pltpu.VMEM((CHUNK, HP), jnp.float32),
            pltpu.VMEM_SHARED((N_PAD, HP), jnp.float32),
            pltpu.SemaphoreType.DMA,
        ],
    )(_segsum_body)


def _segsum(hp, src, dst, zeros):
    return _build_segsum()(hp, src, dst, zeros)


def _inproj_body(h_ref, w_ref, out_ref):
    out_ref[...] = jnp.dot(h_ref[...], w_ref[...],
                           preferred_element_type=jnp.float32)


def _inproj(h_in, w1p):
    return pl.pallas_call(
        _inproj_body,
        out_shape=jax.ShapeDtypeStruct((N, HP), jnp.float32),
    )(h_in, w1p)


def _mlp_body(hp_ref, a0_ref, a1_ref, b1_ref, g_ref, bt_ref, w2_ref, b2_ref,
              wn_ref, out_ref, *, last):
    z = (hp_ref[:, :H] + a0_ref[:, :H] + a1_ref[:, :H]) + b1_ref[...]
    mu = jnp.mean(z, axis=0, keepdims=True)
    zc = z - mu
    var = jnp.mean(zc * zc, axis=0, keepdims=True)
    zn = zc * lax.rsqrt(var + 1e-5) * g_ref[...] + bt_ref[...]
    r = jnp.maximum(zn, 0.0)
    h = jnp.dot(r, w2_ref[...], preferred_element_type=jnp.float32) + b2_ref[...]
    if last:
        out_ref[...] = h
    else:
        h = jnp.maximum(h, 0.0)
        out_ref[...] = jnp.dot(h, wn_ref[...],
                               preferred_element_type=jnp.float32)


def _mlp(hp, a0, a1, b1, g, bt, w2, b2, wn, last):
    width = H if last else HP
    return pl.pallas_call(
        functools.partial(_mlp_body, last=last),
        out_shape=jax.ShapeDtypeStruct((N, width), jnp.float32),
    )(hp, a0, a1, b1, g, bt, w2, b2, wn)


def _pool_body(h_ref, batch_ref, w_ref, b_ref, out_ref):
    bvec = batch_ref[...]
    ids = lax.broadcasted_iota(jnp.int32, (B, N), 0)
    onehot = (bvec == ids).astype(jnp.float32)
    sums = jnp.dot(onehot, h_ref[...], preferred_element_type=jnp.float32)
    counts = jnp.sum(onehot, axis=1, keepdims=True)
    pooled = sums / jnp.maximum(counts, 1.0)
    out_ref[...] = jnp.dot(pooled, w_ref[...],
                           preferred_element_type=jnp.float32) + b_ref[...]


def _pool(h, batch2d, lin_w, lin_b):
    return pl.pallas_call(
        _pool_body,
        out_shape=jax.ShapeDtypeStruct((B, 1), jnp.float32),
    )(h, batch2d, lin_w, lin_b)


def kernel(x, pos, edge_index, batch, params):
    src = edge_index[0]
    dst = edge_index[1]
    zeros = jnp.zeros((N_PAD, HP), jnp.float32)
    h_in = jnp.concatenate([x, pos], axis=1)

    def padw(w):
        return jnp.pad(w, ((0, 0), (0, HP - H)))

    hp = _inproj(h_in, padw(params['layer0']['W1']))
    h_final = None
    for l in range(L):
        p = params[f'layer{l}']
        parts = _segsum(hp, src, dst, zeros)
        a0 = parts[0, :N]
        a1 = parts[1, :N]
        last = l == L - 1
        wn = padw(params[f'layer{l + 1}']['W1']) if not last else p['W2']
        out = _mlp(hp,
                   a0, a1,
                   p['b1'].reshape(1, H),
                   p['gamma'].reshape(1, H),
                   p['beta'].reshape(1, H),
                   p['W2'],
                   p['b2'].reshape(1, H),
                   wn,
                   last)
        if last:
            h_final = out
        else:
            hp = out

    return _pool(h_final, batch.reshape(1, N),
                 params['lin_W'], params['lin_b'].reshape(1, 1))
